# baseline (speedup 1.0000x reference)
import functools

import jax
import jax.numpy as jnp
from jax import lax
from jax.experimental import pallas as pl
from jax.experimental.pallas import tpu as pltpu
from jax.experimental.pallas import tpu_sc as plsc

N = 10000
E = 320000
NUM_EL = 4
H = 128
NB = 8
L = 2
R_MAX = 5.0
LENGTH_CONV = 10.0

NC = 2
NS = 16
NW = NC * NS
EPW = E // NW
CHUNK = 80
IDXD = 4
DATD = 3
RTD = 2
RPW = 624

_SC_MESH = plsc.VectorSubcoreMesh(core_axis_name="c", subcore_axis_name="s")


def _d2_body(src_hbm, dst_hbm, sx_hbm, sy_hbm, sz_hbm,
             px_hbm, py_hbm, pz_hbm, d2_hbm,
             px_v, py_v, pz_v, src_v, dst_v, sx_v, sy_v, sz_v, d2_v):
    c = lax.axis_index("c")
    s = lax.axis_index("s")
    wid = s * NC + c
    base = wid * EPW
    pltpu.sync_copy(px_hbm, px_v)
    pltpu.sync_copy(py_hbm, py_v)
    pltpu.sync_copy(pz_hbm, pz_v)
    pltpu.sync_copy(src_hbm.at[pl.ds(base, EPW)], src_v)
    pltpu.sync_copy(dst_hbm.at[pl.ds(base, EPW)], dst_v)
    pltpu.sync_copy(sx_hbm.at[pl.ds(base, EPW)], sx_v)
    pltpu.sync_copy(sy_hbm.at[pl.ds(base, EPW)], sy_v)
    pltpu.sync_copy(sz_hbm.at[pl.ds(base, EPW)], sz_v)

    def step(i, carry):
        sl = pl.ds(i * 16, 16)
        vs = src_v[sl]
        vd = dst_v[sl]
        dx = plsc.load_gather(px_v, [vd]) - plsc.load_gather(px_v, [vs]) - sx_v[sl]
        dy = plsc.load_gather(py_v, [vd]) - plsc.load_gather(py_v, [vs]) - sy_v[sl]
        dz = plsc.load_gather(pz_v, [vd]) - plsc.load_gather(pz_v, [vs]) - sz_v[sl]
        d2_v[sl] = (dx * dx + dy * dy + dz * dz) * (LENGTH_CONV * LENGTH_CONV)
        return carry

    lax.fori_loop(0, EPW // 16, step, 0)
    pltpu.sync_copy(d2_v, d2_hbm.at[pl.ds(base, EPW)])


@functools.partial(
    pl.kernel,
    out_type=jax.ShapeDtypeStruct((E,), jnp.float32),
    mesh=_SC_MESH,
    scratch_types=[
        pltpu.VMEM((N,), jnp.float32),
        pltpu.VMEM((N,), jnp.float32),
        pltpu.VMEM((N,), jnp.float32),
        pltpu.VMEM((EPW,), jnp.int32),
        pltpu.VMEM((EPW,), jnp.int32),
        pltpu.VMEM((EPW,), jnp.float32),
        pltpu.VMEM((EPW,), jnp.float32),
        pltpu.VMEM((EPW,), jnp.float32),
        pltpu.VMEM((EPW,), jnp.float32),
    ],
    compiler_params=pltpu.CompilerParams(needs_layout_passes=False),
)
def _d2_kernel(*refs):
    _d2_body(*refs)


HH = H // 2
EPT = E // NS
NCHUNK2 = EPT // CHUNK


def _layer_body(ha_hbm, hb_hbm, rth_hbm, src_hbm, dst_hbm, zero_hbm, out_hbm,
                src_v, dst_v, rt_v, hs_v, hd_v, agg_sh,
                isems, rsems, gsems, ssems):
    c = lax.axis_index("c")
    s = lax.axis_index("s")
    pltpu.sync_copy(zero_hbm.at[pl.ds(s * RPW, RPW)], agg_sh.at[pl.ds(s * RPW, RPW)])

    @pl.when(s == NS - 1)
    def _():
        tail = N - NS * RPW
        pltpu.sync_copy(zero_hbm.at[pl.ds(NS * RPW, tail)], agg_sh.at[pl.ds(NS * RPW, tail)])

    plsc.subcore_barrier()

    base0 = s * EPT

    def start_idx(i, ib):
        base = base0 + i * CHUNK
        pltpu.async_copy(src_hbm.at[pl.ds(base, CHUNK)], src_v[ib], isems[ib])
        pltpu.async_copy(dst_hbm.at[pl.ds(base, CHUNK)], dst_v[ib], isems[ib])

    def wait_idx(i, ib):
        base = base0 + i * CHUNK
        pltpu.make_async_copy(src_hbm.at[pl.ds(base, CHUNK)], src_v[ib], isems[ib]).wait()
        pltpu.make_async_copy(dst_hbm.at[pl.ds(base, CHUNK)], dst_v[ib], isems[ib]).wait()

    def start_rt(i, rb):
        base = base0 + i * CHUNK

        @pl.when(c == 0)
        def _():
            pltpu.async_copy(rth_hbm.at[pl.ds(base, CHUNK), pl.ds(0, HH)], rt_v[rb], rsems[rb])

        @pl.when(c == 1)
        def _():
            pltpu.async_copy(rth_hbm.at[pl.ds(base, CHUNK), pl.ds(HH, HH)], rt_v[rb], rsems[rb])

    def wait_rt(i, rb):
        base = base0 + i * CHUNK
        pltpu.make_async_copy(rth_hbm.at[pl.ds(base, CHUNK), pl.ds(0, HH)], rt_v[rb], rsems[rb]).wait()

    def start_gathers(ib, db):
        @pl.when(c == 0)
        def _():
            pltpu.async_copy(ha_hbm.at[src_v[ib]], hs_v[db], gsems[db])
            pltpu.async_copy(ha_hbm.at[dst_v[ib]], hd_v[db], gsems[db])

        @pl.when(c == 1)
        def _():
            pltpu.async_copy(hb_hbm.at[src_v[ib]], hs_v[db], gsems[db])
            pltpu.async_copy(hb_hbm.at[dst_v[ib]], hd_v[db], gsems[db])

    def wait_gathers(ib, db):
        pltpu.make_async_copy(ha_hbm.at[src_v[ib]], hs_v[db], gsems[db]).wait()
        pltpu.make_async_copy(ha_hbm.at[dst_v[ib]], hd_v[db], gsems[db]).wait()

    def compute(db, rb):
        def rows(ri, rcarry):
            for rr in range(2):
                r = ri * 2 + rr
                for cc in range(HH // 16):
                    fsl = pl.ds(cc * 16, 16)
                    rt = rt_v[rb][r, fsl]
                    hs_v[db][r, fsl] = hs_v[db][r, fsl] * rt
                    hd_v[db][r, fsl] = hd_v[db][r, fsl] * rt
            return rcarry

        lax.fori_loop(0, CHUNK // 2, rows, 0)

    def start_scatters(ib, db):
        pltpu.async_copy(hs_v[db], agg_sh.at[dst_v[ib]], ssems[db], add=True)
        pltpu.async_copy(hd_v[db], agg_sh.at[src_v[ib]], ssems[db], add=True)

    def wait_scatters(ib, db):
        pltpu.make_async_copy(hs_v[db], agg_sh.at[dst_v[ib]], ssems[db]).wait()
        pltpu.make_async_copy(hd_v[db], agg_sh.at[src_v[ib]], ssems[db]).wait()

    def step(i, k, scat_wait, has_n1, has_n2):
        i4, i3, i2 = k % IDXD, k % DATD, k % RTD
        n4, n3, n2 = (k + 1) % IDXD, (k + 1) % DATD, (k + 1) % RTD
        if scat_wait:
            wait_scatters((k - 2) % IDXD, (k - 2) % DATD)
        if has_n1:
            wait_idx(i + 1, n4)
            start_gathers(n4, n3)
        if has_n2:
            start_idx(i + 2, (k + 2) % IDXD)
        wait_gathers(i4, i3)
        wait_rt(i, i2)
        compute(i3, i2)
        if has_n1:
            start_rt(i + 1, n2)
        start_scatters(i4, i3)

    start_idx(0, 0)
    start_idx(1, 1)
    start_rt(0, 0)
    wait_idx(0, 0)
    start_gathers(0, 0)
    step(0, 0, scat_wait=False, has_n1=True, has_n2=True)
    step(1, 1, scat_wait=False, has_n1=True, has_n2=True)

    def group(g, carry):
        i0 = 2 + g * 12
        for kk in range(12):
            step(i0 + kk, 2 + kk, scat_wait=True, has_n1=True, has_n2=True)
        return carry

    lax.fori_loop(0, (NCHUNK2 - 2 - 8) // 12, group, 0)

    for i in range(NCHUNK2 - 8, NCHUNK2):
        step(i, i, scat_wait=True, has_n1=(i + 1 < NCHUNK2), has_n2=(i + 2 < NCHUNK2))
    wait_scatters((NCHUNK2 - 2) % IDXD, (NCHUNK2 - 2) % DATD)
    wait_scatters((NCHUNK2 - 1) % IDXD, (NCHUNK2 - 1) % DATD)

    plsc.subcore_barrier()
    pltpu.sync_copy(agg_sh.at[pl.ds(s * RPW, RPW)], out_hbm.at[c, pl.ds(s * RPW, RPW)])

    @pl.when(s == NS - 1)
    def _():
        tail = N - NS * RPW
        pltpu.sync_copy(agg_sh.at[pl.ds(NS * RPW, tail)],
                        out_hbm.at[c, pl.ds(NS * RPW, tail)])


@functools.partial(
    pl.kernel,
    out_type=jax.ShapeDtypeStruct((NC, N, HH), jnp.float32),
    mesh=_SC_MESH,
    scratch_types=[
        [pltpu.VMEM((CHUNK,), jnp.int32) for _ in range(IDXD)],
        [pltpu.VMEM((CHUNK,), jnp.int32) for _ in range(IDXD)],
        [pltpu.VMEM((CHUNK, HH), jnp.float32) for _ in range(RTD)],
        [pltpu.VMEM((CHUNK, HH), jnp.float32) for _ in range(DATD)],
        [pltpu.VMEM((CHUNK, HH), jnp.float32) for _ in range(DATD)],
        pltpu.VMEM_SHARED((N, HH), jnp.float32),
        [pltpu.SemaphoreType.DMA for _ in range(IDXD)],
        [pltpu.SemaphoreType.DMA for _ in range(RTD)],
        [pltpu.SemaphoreType.DMA for _ in range(DATD)],
        [pltpu.SemaphoreType.DMA for _ in range(DATD)],
    ],
    compiler_params=pltpu.CompilerParams(needs_layout_passes=False,
                                         use_tc_tiling_on_sc=False),
)
def _layer_kernel(*refs):
    _layer_body(*refs)


_SBR = 1000
_GR = E // 16


def _rbf_body(d2_ref, z_ref, we_ref, rbf_ref, ha_ref, hb_ref):
    @pl.when(pl.program_id(0) == 0)
    def _():
        z = z_ref[...]
        oh = (z == lax.broadcasted_iota(jnp.int32, (N, NUM_EL), 1)).astype(jnp.float32)
        h = jnp.dot(oh, we_ref[...], preferred_element_type=jnp.float32)
        ha_ref[...] = h[:, :HH]
        hb_ref[...] = h[:, HH:]

    d2 = d2_ref[...]
    rinv = lax.rsqrt(d2 + 1e-12)
    r = d2 * rinv
    u = r * (1.0 / R_MAX)
    u2 = u * u
    u3 = u2 * u
    u6 = u3 * u3
    f = 1.0 - 28.0 * u6 + 48.0 * u6 * u - 21.0 * u6 * u2
    cut = jnp.where(u < 1.0, f, 0.0)
    pref = jnp.sqrt(2.0 / R_MAX) * cut * rinv
    n_lane = (lax.broadcasted_iota(jnp.int32, (_SBR, H), 1) % NB + 1).astype(jnp.float32)
    rbf_ref[...] = jnp.sin(n_lane * (jnp.pi / R_MAX) * r) * pref


def _rbf(d2, atomic_numbers, W_embed):
    d2rep = jnp.broadcast_to(d2[:, None], (E, NB)).reshape(_GR, H)
    full = lambda i: (0, 0)
    return pl.pallas_call(
        _rbf_body,
        grid=(_GR // _SBR,),
        in_specs=[
            pl.BlockSpec((_SBR, H), lambda i: (i, 0)),
            pl.BlockSpec((N, 1), full),
            pl.BlockSpec((NUM_EL, H), full),
        ],
        out_specs=[
            pl.BlockSpec((_SBR, H), lambda i: (i, 0)),
            pl.BlockSpec((N, HH), full),
            pl.BlockSpec((N, HH), full),
        ],
        out_shape=[
            jax.ShapeDtypeStruct((_GR, H), jnp.float32),
            jax.ShapeDtypeStruct((N, HH), jnp.float32),
            jax.ShapeDtypeStruct((N, HH), jnp.float32),
        ],
    )(d2rep, atomic_numbers.astype(jnp.int32).reshape(N, 1), W_embed)


_EB = 8000


def _radial_mlp_body(rbf_ref, w1c_ref, b1c_ref, w2d_ref, rt0_ref, rt1_ref):
    rbf = rbf_ref[...]
    phi = jax.nn.silu(jnp.dot(rbf, w1c_ref[...], preferred_element_type=jnp.float32) + b1c_ref[...])
    rt = jnp.dot(phi, w2d_ref[...], preferred_element_type=jnp.float32)
    rt0_ref[...] = rt[:, :H]
    rt1_ref[...] = rt[:, H:]


def _radial(d2, radial_w1, radial_b1, radial_w2, atomic_numbers, W_embed):
    rbf, ha, hb = _rbf(d2, atomic_numbers, W_embed)
    rbf = rbf.reshape(E, NB)
    w1c = jnp.concatenate([radial_w1[0], radial_w1[1]], axis=1)
    b1c = jnp.concatenate([radial_b1[0], radial_b1[1]], axis=0).reshape(1, 2 * 64)
    w2d = jnp.zeros((H, 2 * H), jnp.float32)
    w2d = w2d.at[:64, :H].set(radial_w2[0]).at[64:, H:].set(radial_w2[1])
    full = lambda i: (0, 0)
    rt0, rt1 = pl.pallas_call(
        _radial_mlp_body,
        grid=(E // _EB,),
        in_specs=[
            pl.BlockSpec((_EB, NB), lambda i: (i, 0)),
            pl.BlockSpec((NB, 2 * 64), full),
            pl.BlockSpec((1, 2 * 64), full),
            pl.BlockSpec((H, 2 * H), full),
        ],
        out_specs=[
            pl.BlockSpec((_EB, H), lambda i: (i, 0)),
            pl.BlockSpec((_EB, H), lambda i: (i, 0)),
        ],
        out_shape=[
            jax.ShapeDtypeStruct((E, H), jnp.float32),
            jax.ShapeDtypeStruct((E, H), jnp.float32),
        ],
    )(rbf, w1c, b1c, w2d)
    return rt0, rt1, ha, hb


def _update_body(agg_ref, ha_ref, hb_ref, wm_ref, wr_ref, hna_ref, hnb_ref, e_ref):
    agg = jnp.concatenate([agg_ref[0], agg_ref[1]], axis=1)
    h = jnp.concatenate([ha_ref[...], hb_ref[...]], axis=1)
    hn = jax.nn.silu(jnp.dot(agg, wm_ref[...], preferred_element_type=jnp.float32)) + h
    hna_ref[...] = hn[:, :HH]
    hnb_ref[...] = hn[:, HH:]
    e_ref[...] = jnp.sum(hn * wr_ref[...])[None, None]


def _update(aggh, ha, hb, Wm, Wr):
    return pl.pallas_call(
        _update_body,
        out_shape=[
            jax.ShapeDtypeStruct((N, HH), jnp.float32),
            jax.ShapeDtypeStruct((N, HH), jnp.float32),
            jax.ShapeDtypeStruct((1, 1), jnp.float32),
        ],
    )(aggh, ha, hb, Wm, Wr.reshape(1, H))


def kernel(positions, atomic_numbers, pairs, shifts, W_embed, radial_w1,
           radial_b1, radial_w2, W_msg, W_read):
    src = jnp.copy(pairs[:, 0].astype(jnp.int32))
    dst = jnp.copy(pairs[:, 1].astype(jnp.int32))
    sx = jnp.copy(shifts[:, 0])
    sy = jnp.copy(shifts[:, 1])
    sz = jnp.copy(shifts[:, 2])
    px = jnp.copy(positions[:, 0])
    py = jnp.copy(positions[:, 1])
    pz = jnp.copy(positions[:, 2])

    d2 = _d2_kernel(src, dst, sx, sy, sz, px, py, pz)
    rt0, rt1, ha, hb = _radial(d2, radial_w1, radial_b1, radial_w2,
                               atomic_numbers, W_embed)
    zero = jnp.zeros((N, HH), jnp.float32)

    energy = jnp.zeros((), jnp.float32)
    for t, rt in enumerate((rt0, rt1)):
        aggh = _layer_kernel(ha, hb, rt, src, dst, zero)
        ha, hb, e = _update(aggh, ha, hb, W_msg[t], W_read[t])
        energy = energy + e[0, 0]
    return energy.reshape(1)


# speedup vs baseline: 1.0115x; 1.0115x over previous
import functools

import jax
import jax.numpy as jnp
from jax import lax
from jax.experimental import pallas as pl
from jax.experimental.pallas import tpu as pltpu
from jax.experimental.pallas import tpu_sc as plsc

N = 10000
E = 320000
NUM_EL = 4
H = 128
NB = 8
L = 2
R_MAX = 5.0
LENGTH_CONV = 10.0

NC = 2
NS = 16
NW = NC * NS
EPW = E // NW
CHUNK = 80
IDXD = 4
DATD = 3
RTD = 2
RPW = 624

_SC_MESH = plsc.VectorSubcoreMesh(core_axis_name="c", subcore_axis_name="s")


def _d2_body(src_hbm, dst_hbm, sx_hbm, sy_hbm, sz_hbm,
             px_hbm, py_hbm, pz_hbm, d2_hbm,
             px_v, py_v, pz_v, src_v, dst_v, sx_v, sy_v, sz_v, d2_v):
    c = lax.axis_index("c")
    s = lax.axis_index("s")
    wid = s * NC + c
    base = wid * EPW
    pltpu.sync_copy(px_hbm, px_v)
    pltpu.sync_copy(py_hbm, py_v)
    pltpu.sync_copy(pz_hbm, pz_v)
    pltpu.sync_copy(src_hbm.at[pl.ds(base, EPW)], src_v)
    pltpu.sync_copy(dst_hbm.at[pl.ds(base, EPW)], dst_v)
    pltpu.sync_copy(sx_hbm.at[pl.ds(base, EPW)], sx_v)
    pltpu.sync_copy(sy_hbm.at[pl.ds(base, EPW)], sy_v)
    pltpu.sync_copy(sz_hbm.at[pl.ds(base, EPW)], sz_v)

    def step(i, carry):
        sl = pl.ds(i * 16, 16)
        vs = src_v[sl]
        vd = dst_v[sl]
        dx = plsc.load_gather(px_v, [vd]) - plsc.load_gather(px_v, [vs]) - sx_v[sl]
        dy = plsc.load_gather(py_v, [vd]) - plsc.load_gather(py_v, [vs]) - sy_v[sl]
        dz = plsc.load_gather(pz_v, [vd]) - plsc.load_gather(pz_v, [vs]) - sz_v[sl]
        d2_v[sl] = (dx * dx + dy * dy + dz * dz) * (LENGTH_CONV * LENGTH_CONV)
        return carry

    lax.fori_loop(0, EPW // 16, step, 0)
    pltpu.sync_copy(d2_v, d2_hbm.at[pl.ds(base, EPW)])


@functools.partial(
    pl.kernel,
    out_type=jax.ShapeDtypeStruct((E,), jnp.float32),
    mesh=_SC_MESH,
    scratch_types=[
        pltpu.VMEM((N,), jnp.float32),
        pltpu.VMEM((N,), jnp.float32),
        pltpu.VMEM((N,), jnp.float32),
        pltpu.VMEM((EPW,), jnp.int32),
        pltpu.VMEM((EPW,), jnp.int32),
        pltpu.VMEM((EPW,), jnp.float32),
        pltpu.VMEM((EPW,), jnp.float32),
        pltpu.VMEM((EPW,), jnp.float32),
        pltpu.VMEM((EPW,), jnp.float32),
    ],
    compiler_params=pltpu.CompilerParams(needs_layout_passes=False),
)
def _d2_kernel(*refs):
    _d2_body(*refs)


HH = H // 2
EPT = E // NS
NCHUNK2 = EPT // CHUNK


def _layer_body(ha_hbm, hb_hbm, rth_hbm, src_hbm, dst_hbm, zero_hbm, out_hbm,
                src_v, dst_v, rt_v, hs_v, hd_v, agg_sh,
                isems, rsems, gsems, ssems):
    c = lax.axis_index("c")
    s = lax.axis_index("s")
    pltpu.sync_copy(zero_hbm.at[pl.ds(s * RPW, RPW)], agg_sh.at[pl.ds(s * RPW, RPW)])

    @pl.when(s == NS - 1)
    def _():
        tail = N - NS * RPW
        pltpu.sync_copy(zero_hbm.at[pl.ds(NS * RPW, tail)], agg_sh.at[pl.ds(NS * RPW, tail)])

    plsc.subcore_barrier()

    base0 = s * EPT

    def start_idx(i, ib):
        base = base0 + i * CHUNK
        pltpu.async_copy(src_hbm.at[pl.ds(base, CHUNK)], src_v[ib], isems[ib])
        pltpu.async_copy(dst_hbm.at[pl.ds(base, CHUNK)], dst_v[ib], isems[ib])

    def wait_idx(i, ib):
        base = base0 + i * CHUNK
        pltpu.make_async_copy(src_hbm.at[pl.ds(base, CHUNK)], src_v[ib], isems[ib]).wait()
        pltpu.make_async_copy(dst_hbm.at[pl.ds(base, CHUNK)], dst_v[ib], isems[ib]).wait()

    def start_rt(i, rb):
        base = base0 + i * CHUNK

        @pl.when(c == 0)
        def _():
            pltpu.async_copy(rth_hbm.at[pl.ds(base, CHUNK), pl.ds(0, HH)], rt_v[rb], rsems[rb])

        @pl.when(c == 1)
        def _():
            pltpu.async_copy(rth_hbm.at[pl.ds(base, CHUNK), pl.ds(HH, HH)], rt_v[rb], rsems[rb])

    def wait_rt(i, rb):
        base = base0 + i * CHUNK
        pltpu.make_async_copy(rth_hbm.at[pl.ds(base, CHUNK), pl.ds(0, HH)], rt_v[rb], rsems[rb]).wait()

    def start_gathers(ib, db):
        @pl.when(c == 0)
        def _():
            pltpu.async_copy(ha_hbm.at[src_v[ib]], hs_v[db], gsems[db])
            pltpu.async_copy(ha_hbm.at[dst_v[ib]], hd_v[db], gsems[db])

        @pl.when(c == 1)
        def _():
            pltpu.async_copy(hb_hbm.at[src_v[ib]], hs_v[db], gsems[db])
            pltpu.async_copy(hb_hbm.at[dst_v[ib]], hd_v[db], gsems[db])

    def wait_gathers(ib, db):
        pltpu.make_async_copy(ha_hbm.at[src_v[ib]], hs_v[db], gsems[db]).wait()
        pltpu.make_async_copy(ha_hbm.at[dst_v[ib]], hd_v[db], gsems[db]).wait()

    def compute(db, rb):
        def rows(ri, rcarry):
            for rr in range(2):
                r = ri * 2 + rr
                for cc in range(HH // 16):
                    fsl = pl.ds(cc * 16, 16)
                    rt = rt_v[rb][r, fsl]
                    hs_v[db][r, fsl] = hs_v[db][r, fsl] * rt
                    hd_v[db][r, fsl] = hd_v[db][r, fsl] * rt
            return rcarry

        lax.fori_loop(0, CHUNK // 2, rows, 0)

    def start_scatters(ib, db):
        pltpu.async_copy(hs_v[db], agg_sh.at[dst_v[ib]], ssems[db], add=True)
        pltpu.async_copy(hd_v[db], agg_sh.at[src_v[ib]], ssems[db], add=True)

    def wait_scatters(ib, db):
        pltpu.make_async_copy(hs_v[db], agg_sh.at[dst_v[ib]], ssems[db]).wait()
        pltpu.make_async_copy(hd_v[db], agg_sh.at[src_v[ib]], ssems[db]).wait()

    def step(i, k, scat_wait, has_n1, has_n2):
        i4, i3, i2 = k % IDXD, k % DATD, k % RTD
        n4, n3, n2 = (k + 1) % IDXD, (k + 1) % DATD, (k + 1) % RTD
        if scat_wait:
            wait_scatters((k - 2) % IDXD, (k - 2) % DATD)
        if has_n1:
            wait_idx(i + 1, n4)
            start_gathers(n4, n3)
        if has_n2:
            start_idx(i + 2, (k + 2) % IDXD)
        wait_gathers(i4, i3)
        wait_rt(i, i2)
        compute(i3, i2)
        if has_n1:
            start_rt(i + 1, n2)
        start_scatters(i4, i3)

    start_idx(0, 0)
    start_idx(1, 1)
    start_rt(0, 0)
    wait_idx(0, 0)
    start_gathers(0, 0)
    step(0, 0, scat_wait=False, has_n1=True, has_n2=True)
    step(1, 1, scat_wait=False, has_n1=True, has_n2=True)

    def group(g, carry):
        i0 = 2 + g * 12
        for kk in range(12):
            step(i0 + kk, 2 + kk, scat_wait=True, has_n1=True, has_n2=True)
        return carry

    lax.fori_loop(0, (NCHUNK2 - 2 - 8) // 12, group, 0)

    for i in range(NCHUNK2 - 8, NCHUNK2):
        step(i, i, scat_wait=True, has_n1=(i + 1 < NCHUNK2), has_n2=(i + 2 < NCHUNK2))
    wait_scatters((NCHUNK2 - 2) % IDXD, (NCHUNK2 - 2) % DATD)
    wait_scatters((NCHUNK2 - 1) % IDXD, (NCHUNK2 - 1) % DATD)

    plsc.subcore_barrier()
    pltpu.sync_copy(agg_sh.at[pl.ds(s * RPW, RPW)], out_hbm.at[c, pl.ds(s * RPW, RPW)])

    @pl.when(s == NS - 1)
    def _():
        tail = N - NS * RPW
        pltpu.sync_copy(agg_sh.at[pl.ds(NS * RPW, tail)],
                        out_hbm.at[c, pl.ds(NS * RPW, tail)])


@functools.partial(
    pl.kernel,
    out_type=jax.ShapeDtypeStruct((NC, N, HH), jnp.float32),
    mesh=_SC_MESH,
    scratch_types=[
        [pltpu.VMEM((CHUNK,), jnp.int32) for _ in range(IDXD)],
        [pltpu.VMEM((CHUNK,), jnp.int32) for _ in range(IDXD)],
        [pltpu.VMEM((CHUNK, HH), jnp.float32) for _ in range(RTD)],
        [pltpu.VMEM((CHUNK, HH), jnp.float32) for _ in range(DATD)],
        [pltpu.VMEM((CHUNK, HH), jnp.float32) for _ in range(DATD)],
        pltpu.VMEM_SHARED((N, HH), jnp.float32),
        [pltpu.SemaphoreType.DMA for _ in range(IDXD)],
        [pltpu.SemaphoreType.DMA for _ in range(RTD)],
        [pltpu.SemaphoreType.DMA for _ in range(DATD)],
        [pltpu.SemaphoreType.DMA for _ in range(DATD)],
    ],
    compiler_params=pltpu.CompilerParams(needs_layout_passes=False,
                                         use_tc_tiling_on_sc=False),
)
def _layer_kernel(*refs):
    _layer_body(*refs)


_SBR = 1000
_GR = E // 16


def _rbf_body(d2_ref, rbf_ref):
    d2 = d2_ref[...]
    rinv = lax.rsqrt(d2 + 1e-12)
    r = d2 * rinv
    u = r * (1.0 / R_MAX)
    u2 = u * u
    u3 = u2 * u
    u6 = u3 * u3
    f = 1.0 - 28.0 * u6 + 48.0 * u6 * u - 21.0 * u6 * u2
    cut = jnp.where(u < 1.0, f, 0.0)
    pref = jnp.sqrt(2.0 / R_MAX) * cut * rinv
    n_lane = (lax.broadcasted_iota(jnp.int32, (_SBR, H), 1) % NB + 1).astype(jnp.float32)
    rbf_ref[...] = jnp.sin(n_lane * (jnp.pi / R_MAX) * r) * pref


def _rbf(d2):
    d2rep = jnp.broadcast_to(d2[:, None], (E, NB)).reshape(_GR, H)
    return pl.pallas_call(
        _rbf_body,
        grid=(_GR // _SBR,),
        in_specs=[pl.BlockSpec((_SBR, H), lambda i: (i, 0))],
        out_specs=pl.BlockSpec((_SBR, H), lambda i: (i, 0)),
        out_shape=jax.ShapeDtypeStruct((_GR, H), jnp.float32),
    )(d2rep)


_EB = 8000


def _radial_mlp_body(rbf_ref, w1c_ref, b1c_ref, w2d_ref, rt0_ref, rt1_ref):
    rbf = rbf_ref[...]
    phi = jax.nn.silu(jnp.dot(rbf, w1c_ref[...], preferred_element_type=jnp.float32) + b1c_ref[...])
    rt = jnp.dot(phi, w2d_ref[...], preferred_element_type=jnp.float32)
    rt0_ref[...] = rt[:, :H]
    rt1_ref[...] = rt[:, H:]


def _radial(d2, radial_w1, radial_b1, radial_w2):
    rbf = _rbf(d2).reshape(E, NB)
    w1c = jnp.concatenate([radial_w1[0], radial_w1[1]], axis=1)
    b1c = jnp.concatenate([radial_b1[0], radial_b1[1]], axis=0).reshape(1, 2 * 64)
    w2d = jnp.zeros((H, 2 * H), jnp.float32)
    w2d = w2d.at[:64, :H].set(radial_w2[0]).at[64:, H:].set(radial_w2[1])
    full = lambda i: (0, 0)
    rt0, rt1 = pl.pallas_call(
        _radial_mlp_body,
        grid=(E // _EB,),
        in_specs=[
            pl.BlockSpec((_EB, NB), lambda i: (i, 0)),
            pl.BlockSpec((NB, 2 * 64), full),
            pl.BlockSpec((1, 2 * 64), full),
            pl.BlockSpec((H, 2 * H), full),
        ],
        out_specs=[
            pl.BlockSpec((_EB, H), lambda i: (i, 0)),
            pl.BlockSpec((_EB, H), lambda i: (i, 0)),
        ],
        out_shape=[
            jax.ShapeDtypeStruct((E, H), jnp.float32),
            jax.ShapeDtypeStruct((E, H), jnp.float32),
        ],
    )(rbf, w1c, b1c, w2d)
    return rt0, rt1


def _embed_body(z_ref, we_ref, ha_ref, hb_ref):
    z = z_ref[...]
    oh = (z == lax.broadcasted_iota(jnp.int32, (N, NUM_EL), 1)).astype(jnp.float32)
    h = jnp.dot(oh, we_ref[...], preferred_element_type=jnp.float32)
    ha_ref[...] = h[:, :HH]
    hb_ref[...] = h[:, HH:]


def _embed(atomic_numbers, W_embed):
    return pl.pallas_call(
        _embed_body,
        out_shape=[
            jax.ShapeDtypeStruct((N, HH), jnp.float32),
            jax.ShapeDtypeStruct((N, HH), jnp.float32),
        ],
    )(atomic_numbers.astype(jnp.int32).reshape(N, 1), W_embed)


def _update_body(agg_ref, ha_ref, hb_ref, wm_ref, wr_ref, hna_ref, hnb_ref, e_ref):
    agg = jnp.concatenate([agg_ref[0], agg_ref[1]], axis=1)
    h = jnp.concatenate([ha_ref[...], hb_ref[...]], axis=1)
    hn = jax.nn.silu(jnp.dot(agg, wm_ref[...], preferred_element_type=jnp.float32)) + h
    hna_ref[...] = hn[:, :HH]
    hnb_ref[...] = hn[:, HH:]
    e_ref[...] = jnp.sum(hn * wr_ref[...])[None, None]


def _update(aggh, ha, hb, Wm, Wr):
    return pl.pallas_call(
        _update_body,
        out_shape=[
            jax.ShapeDtypeStruct((N, HH), jnp.float32),
            jax.ShapeDtypeStruct((N, HH), jnp.float32),
            jax.ShapeDtypeStruct((1, 1), jnp.float32),
        ],
    )(aggh, ha, hb, Wm, Wr.reshape(1, H))


def kernel(positions, atomic_numbers, pairs, shifts, W_embed, radial_w1,
           radial_b1, radial_w2, W_msg, W_read):
    src = jnp.copy(pairs[:, 0].astype(jnp.int32))
    dst = jnp.copy(pairs[:, 1].astype(jnp.int32))
    sx = jnp.copy(shifts[:, 0])
    sy = jnp.copy(shifts[:, 1])
    sz = jnp.copy(shifts[:, 2])
    px = jnp.copy(positions[:, 0])
    py = jnp.copy(positions[:, 1])
    pz = jnp.copy(positions[:, 2])

    d2 = _d2_kernel(src, dst, sx, sy, sz, px, py, pz)
    rt0, rt1 = _radial(d2, radial_w1, radial_b1, radial_w2)
    ha, hb = _embed(atomic_numbers, W_embed)
    zero = jnp.zeros((N, HH), jnp.float32)

    energy = jnp.zeros((), jnp.float32)
    for t, rt in enumerate((rt0, rt1)):
        aggh = _layer_kernel(ha, hb, rt, src, dst, zero)
        ha, hb, e = _update(aggh, ha, hb, W_msg[t], W_read[t])
        energy = energy + e[0, 0]
    return energy.reshape(1)
